# Initial kernel scaffold; baseline (speedup 1.0000x reference)
import jax
import jax.numpy as jnp
from jax.experimental import pallas as pl


def kernel(node_feat, node_attr, edge_idx, edge_dist_embedding, edge_diff_embedding, W_skip, W_lin1, W_mlp0, W_mlp1, W_mlp2, W_mlp3, W2_0, W2_1, W2_2):
    raise NotImplementedError("write your pallas kernel here")


# speedup vs baseline: 4.3461x; 4.3461x over previous
import functools
import math

import jax
import jax.numpy as jnp
from jax import lax
from jax.experimental import pallas as pl
from jax.experimental.pallas import tpu as pltpu
from jax.experimental.pallas import tpu_sc as plsc

MUL = 128
HID = 64
NATTR = 10
DGRP = 2
NPASS = 5
GW = DGRP * HID
AVG_NUM_NEIGHBORS = 32.0


def _mlp_g_body(ed_ref, sh_ref, w0_ref, w1_ref, w2_ref, *g_refs):
    h = jax.nn.silu(jnp.dot(ed_ref[...], w0_ref[...],
                            preferred_element_type=jnp.float32) * (1.0 / math.sqrt(8.0)))
    h = jax.nn.silu(jnp.dot(h, w1_ref[...],
                            preferred_element_type=jnp.float32) * (1.0 / math.sqrt(HID)))
    h = jax.nn.silu(jnp.dot(h, w2_ref[...],
                            preferred_element_type=jnp.float32) * (1.0 / math.sqrt(HID)))
    sh = sh_ref[...]
    for p, ref in enumerate(g_refs):
        cols = []
        for d in range(DGRP):
            dc = DGRP * p + d
            cols.append(sh[:, dc:dc + 1] * h if dc < 9 else jnp.zeros_like(h))
        ref[...] = jnp.concatenate(cols, axis=1)


def _build_sc_scatter(E, N):
    n_tiles = 32
    ept = E // n_tiles
    C = 80
    n_chunks = ept // C
    rows = (N + 127) // 128 * 128 + 128
    rows_per_tile = rows // 16
    mesh = plsc.VectorSubcoreMesh(core_axis_name="c", subcore_axis_name="s")
    out_t = tuple(jax.ShapeDtypeStruct((2, rows, GW), jnp.float32) for _ in range(NPASS))

    @functools.partial(
        pl.kernel, out_type=out_t, mesh=mesh,
        scratch_types=[
            pltpu.VMEM((C,), jnp.int32),
            pltpu.VMEM((C, GW), jnp.float32),
            pltpu.VMEM_SHARED((rows, GW), jnp.float32),
        ])
    def sc_scatter(src_hbm, g0_hbm, g1_hbm, g2_hbm, g3_hbm, g4_hbm, zero_hbm,
                   o0, o1, o2, o3, o4, idx_v, rows_v, accum):
        c = lax.axis_index("c")
        s = lax.axis_index("s")
        tile_base = (c * 16 + s) * ept
        row0 = s * rows_per_tile
        for g_hbm, o_hbm in ((g0_hbm, o0), (g1_hbm, o1), (g2_hbm, o2),
                             (g3_hbm, o3), (g4_hbm, o4)):
            pltpu.sync_copy(zero_hbm.at[pl.ds(row0, rows_per_tile)],
                            accum.at[pl.ds(row0, rows_per_tile)])
            plsc.subcore_barrier()

            def chunk(i, carry):
                start = tile_base + i * C
                pltpu.sync_copy(src_hbm.at[pl.ds(start, C)], idx_v)
                pltpu.sync_copy(g_hbm.at[pl.ds(start, C)], rows_v)
                pltpu.sync_copy(rows_v, accum.at[idx_v], add=True)
                return carry

            lax.fori_loop(0, n_chunks, chunk, 0)
            plsc.subcore_barrier()
            pltpu.sync_copy(accum.at[pl.ds(row0, rows_per_tile)],
                            o_hbm.at[c, pl.ds(row0, rows_per_tile)])

    return sc_scatter


def _final_body(nf_ref, na_ref, *rest):
    t_refs = rest[:2 * NPASS]
    wskip_ref, wlin_ref, wmlp3_ref, w20_ref, w21_ref, w22_ref, out9_ref, sc_ref = rest[2 * NPASS:]
    nf = nf_ref[...]
    acc = jnp.zeros_like(nf)
    for v in range(NATTR):
        acc = acc + jnp.dot(nf, wskip_ref[:, v, :],
                            preferred_element_type=jnp.float32) * na_ref[:, v:v + 1]
    sc_ref[...] = acc * (1.0 / math.sqrt(MUL * NATTR))
    x = jnp.dot(nf, wlin_ref[...],
                preferred_element_type=jnp.float32) * (1.0 / math.sqrt(MUL))
    w2s = (w20_ref, w21_ref, w22_ref)
    scale = 1.0 / (math.sqrt(HID) * math.sqrt(MUL) * AVG_NUM_NEIGHBORS)
    for d in range(9):
        p, dl = divmod(d, DGRP)
        l = 0 if d == 0 else (1 if d <= 3 else 2)
        ta, tb = t_refs[2 * p], t_refs[2 * p + 1]
        td = ta[:, HID * dl:HID * (dl + 1)] + tb[:, HID * dl:HID * (dl + 1)]
        m = jnp.dot(td, wmlp3_ref[:, l * MUL:(l + 1) * MUL],
                    preferred_element_type=jnp.float32)
        out9_ref[d] = jnp.dot(x * m, w2s[l][...],
                              preferred_element_type=jnp.float32) * scale


def kernel(node_feat, node_attr, edge_idx, edge_dist_embedding, edge_diff_embedding,
           W_skip, W_lin1, W_mlp0, W_mlp1, W_mlp2, W_mlp3, W2_0, W2_1, W2_2):
    N = node_feat.shape[0]
    E = edge_dist_embedding.shape[0]
    src = edge_idx[:, 0]

    EB = 2560
    gs = pl.pallas_call(
        _mlp_g_body,
        grid=(E // EB,),
        in_specs=[
            pl.BlockSpec((EB, 8), lambda i: (i, 0)),
            pl.BlockSpec((EB, 9), lambda i: (i, 0)),
            pl.BlockSpec((8, HID), lambda i: (0, 0)),
            pl.BlockSpec((HID, HID), lambda i: (0, 0)),
            pl.BlockSpec((HID, HID), lambda i: (0, 0)),
        ],
        out_specs=[pl.BlockSpec((EB, GW), lambda i: (i, 0))] * NPASS,
        out_shape=[jax.ShapeDtypeStruct((E, GW), jnp.float32)] * NPASS,
    )(edge_dist_embedding, edge_diff_embedding, W_mlp0, W_mlp1, W_mlp2)

    rows = (N + 127) // 128 * 128 + 128
    zeros = jnp.zeros((rows, GW), jnp.float32)
    ts = _build_sc_scatter(E, N)(src, *gs, zeros)
    t_halves = [t[c] for t in ts for c in range(2)]

    NB = 400
    out9, sc = pl.pallas_call(
        _final_body,
        grid=(N // NB,),
        in_specs=[
            pl.BlockSpec((NB, MUL), lambda i: (i, 0)),
            pl.BlockSpec((NB, NATTR), lambda i: (i, 0)),
        ] + [pl.BlockSpec((NB, GW), lambda i: (i, 0))] * (2 * NPASS) + [
            pl.BlockSpec((MUL, NATTR, MUL), lambda i: (0, 0, 0)),
            pl.BlockSpec((MUL, MUL), lambda i: (0, 0)),
            pl.BlockSpec((HID, 3 * MUL), lambda i: (0, 0)),
            pl.BlockSpec((MUL, MUL), lambda i: (0, 0)),
            pl.BlockSpec((MUL, MUL), lambda i: (0, 0)),
            pl.BlockSpec((MUL, MUL), lambda i: (0, 0)),
        ],
        out_specs=[
            pl.BlockSpec((9, NB, MUL), lambda i: (0, i, 0)),
            pl.BlockSpec((NB, MUL), lambda i: (i, 0)),
        ],
        out_shape=[
            jax.ShapeDtypeStruct((9, N, MUL), jnp.float32),
            jax.ShapeDtypeStruct((N, MUL), jnp.float32),
        ],
    )(node_feat, node_attr, *t_halves,
      W_skip, W_lin1, W_mlp3, W2_0, W2_1, W2_2)

    return (jnp.transpose(out9, (1, 2, 0)), sc)


# speedup vs baseline: 6.9386x; 1.5965x over previous
import functools
import math

import jax
import jax.numpy as jnp
from jax import lax
from jax.experimental import pallas as pl
from jax.experimental.pallas import tpu as pltpu
from jax.experimental.pallas import tpu_sc as plsc

MUL = 128
HID = 64
NATTR = 10
DGRP = 2
NPASS = 5
GW = DGRP * HID
AVG_NUM_NEIGHBORS = 32.0


def _mlp_g_body(ed_ref, sh_ref, w0_ref, w1_ref, w2_ref, *g_refs):
    h = jax.nn.silu(jnp.dot(ed_ref[...], w0_ref[...],
                            preferred_element_type=jnp.float32) * (1.0 / math.sqrt(8.0)))
    h = jax.nn.silu(jnp.dot(h, w1_ref[...],
                            preferred_element_type=jnp.float32) * (1.0 / math.sqrt(HID)))
    h = jax.nn.silu(jnp.dot(h, w2_ref[...],
                            preferred_element_type=jnp.float32) * (1.0 / math.sqrt(HID)))
    sh = sh_ref[...]
    for p, ref in enumerate(g_refs):
        cols = []
        for d in range(DGRP):
            dc = DGRP * p + d
            cols.append(sh[:, dc:dc + 1] * h if dc < 9 else jnp.zeros_like(h))
        ref[...] = jnp.concatenate(cols, axis=1)


def _build_sc_scatter(E, N):
    n_tiles = 32
    ept = E // n_tiles
    C = 40
    n_chunks = ept // C
    NBUF = 5
    n_outer = n_chunks // NBUF
    rows = (N + 127) // 128 * 128 + 128
    rows_per_tile = rows // 16
    mesh = plsc.VectorSubcoreMesh(core_axis_name="c", subcore_axis_name="s")
    out_t = tuple(jax.ShapeDtypeStruct((2, rows, GW), jnp.float32) for _ in range(NPASS))

    @functools.partial(
        pl.kernel, out_type=out_t, mesh=mesh,
        scratch_types=[pltpu.VMEM((C, GW), jnp.float32)] * NBUF
                      + [pltpu.VMEM((C,), jnp.int32)] * NBUF + [
            pltpu.VMEM_SHARED((rows, GW), jnp.float32),
        ] + [pltpu.SemaphoreType.DMA] * NBUF)
    def sc_scatter(src_hbm, g0_hbm, g1_hbm, g2_hbm, g3_hbm, g4_hbm, zero_hbm,
                   o0, o1, o2, o3, o4, *rest):
        bufs = rest[:NBUF]
        idxs = rest[NBUF:2 * NBUF]
        accum = rest[2 * NBUF]
        sems = rest[2 * NBUF + 1:]
        c = lax.axis_index("c")
        s = lax.axis_index("s")
        wid = c * 16 + s
        tile_base = wid * ept
        row0 = s * rows_per_tile

        for g_hbm, o_hbm in ((g0_hbm, o0), (g1_hbm, o1), (g2_hbm, o2),
                             (g3_hbm, o3), (g4_hbm, o4)):
            pltpu.sync_copy(zero_hbm.at[pl.ds(row0, rows_per_tile)],
                            accum.at[pl.ds(row0, rows_per_tile)])
            plsc.subcore_barrier()

            def fetch(i, b):
                pltpu.async_copy(src_hbm.at[wid, i], idxs[b], sems[b])
                pltpu.async_copy(g_hbm.at[pl.ds(tile_base + i * C, C)],
                                 bufs[b], sems[b])

            for b in range(NBUF):
                fetch(b, b)

            def outer(j, carry):
                for b in range(NBUF):
                    i = j * NBUF + b
                    pltpu.make_async_copy(src_hbm.at[0, 0], idxs[b],
                                          sems[b]).wait()
                    pltpu.make_async_copy(g_hbm.at[pl.ds(0, C)], bufs[b],
                                          sems[b]).wait()
                    pltpu.sync_copy(bufs[b], accum.at[idxs[b]], add=True)

                    @pl.when(j < n_outer - 1)
                    def _():
                        fetch(i + NBUF, b)
                return carry

            lax.fori_loop(0, n_outer, outer, 0)
            plsc.subcore_barrier()
            pltpu.sync_copy(accum.at[pl.ds(row0, rows_per_tile)],
                            o_hbm.at[c, pl.ds(row0, rows_per_tile)])

    return sc_scatter


def _final_body(nf_ref, na_ref, *rest):
    t_refs = rest[:2 * NPASS]
    wskip_ref, wlin_ref, wmlp3_ref, w20_ref, w21_ref, w22_ref, out9_ref, sc_ref = rest[2 * NPASS:]
    nf = nf_ref[...]
    acc = jnp.zeros_like(nf)
    for v in range(NATTR):
        acc = acc + jnp.dot(nf, wskip_ref[:, v, :],
                            preferred_element_type=jnp.float32) * na_ref[:, v:v + 1]
    sc_ref[...] = acc * (1.0 / math.sqrt(MUL * NATTR))
    x = jnp.dot(nf, wlin_ref[...],
                preferred_element_type=jnp.float32) * (1.0 / math.sqrt(MUL))
    w2s = (w20_ref, w21_ref, w22_ref)
    scale = 1.0 / (math.sqrt(HID) * math.sqrt(MUL) * AVG_NUM_NEIGHBORS)
    for d in range(9):
        p, dl = divmod(d, DGRP)
        l = 0 if d == 0 else (1 if d <= 3 else 2)
        ta, tb = t_refs[2 * p], t_refs[2 * p + 1]
        td = ta[:, HID * dl:HID * (dl + 1)] + tb[:, HID * dl:HID * (dl + 1)]
        m = jnp.dot(td, wmlp3_ref[:, l * MUL:(l + 1) * MUL],
                    preferred_element_type=jnp.float32)
        out9_ref[d] = jnp.dot(x * m, w2s[l][...],
                              preferred_element_type=jnp.float32) * scale


def kernel(node_feat, node_attr, edge_idx, edge_dist_embedding, edge_diff_embedding,
           W_skip, W_lin1, W_mlp0, W_mlp1, W_mlp2, W_mlp3, W2_0, W2_1, W2_2):
    N = node_feat.shape[0]
    E = edge_dist_embedding.shape[0]
    src = edge_idx[:, 0]

    EB = 2560
    gs = pl.pallas_call(
        _mlp_g_body,
        grid=(E // EB,),
        in_specs=[
            pl.BlockSpec((EB, 8), lambda i: (i, 0)),
            pl.BlockSpec((EB, 9), lambda i: (i, 0)),
            pl.BlockSpec((8, HID), lambda i: (0, 0)),
            pl.BlockSpec((HID, HID), lambda i: (0, 0)),
            pl.BlockSpec((HID, HID), lambda i: (0, 0)),
        ],
        out_specs=[pl.BlockSpec((EB, GW), lambda i: (i, 0))] * NPASS,
        out_shape=[jax.ShapeDtypeStruct((E, GW), jnp.float32)] * NPASS,
    )(edge_dist_embedding, edge_diff_embedding, W_mlp0, W_mlp1, W_mlp2)

    rows = (N + 127) // 128 * 128 + 128
    zeros = jnp.zeros((rows, GW), jnp.float32)
    ts = _build_sc_scatter(E, N)(src.reshape(32, -1, 40), *gs, zeros)
    t_halves = [t[c] for t in ts for c in range(2)]

    NB = 400
    out9, sc = pl.pallas_call(
        _final_body,
        grid=(N // NB,),
        in_specs=[
            pl.BlockSpec((NB, MUL), lambda i: (i, 0)),
            pl.BlockSpec((NB, NATTR), lambda i: (i, 0)),
        ] + [pl.BlockSpec((NB, GW), lambda i: (i, 0))] * (2 * NPASS) + [
            pl.BlockSpec((MUL, NATTR, MUL), lambda i: (0, 0, 0)),
            pl.BlockSpec((MUL, MUL), lambda i: (0, 0)),
            pl.BlockSpec((HID, 3 * MUL), lambda i: (0, 0)),
            pl.BlockSpec((MUL, MUL), lambda i: (0, 0)),
            pl.BlockSpec((MUL, MUL), lambda i: (0, 0)),
            pl.BlockSpec((MUL, MUL), lambda i: (0, 0)),
        ],
        out_specs=[
            pl.BlockSpec((9, NB, MUL), lambda i: (0, i, 0)),
            pl.BlockSpec((NB, MUL), lambda i: (i, 0)),
        ],
        out_shape=[
            jax.ShapeDtypeStruct((9, N, MUL), jnp.float32),
            jax.ShapeDtypeStruct((N, MUL), jnp.float32),
        ],
    )(node_feat, node_attr, *t_halves,
      W_skip, W_lin1, W_mlp3, W2_0, W2_1, W2_2)

    return (jnp.transpose(out9, (1, 2, 0)), sc)
